# Initial kernel scaffold; baseline (speedup 1.0000x reference)
import jax
import jax.numpy as jnp
from jax.experimental import pallas as pl


def kernel(x, tok_table, pos_table):
    raise NotImplementedError("write your pallas kernel here")


# speedup vs baseline: 2.4484x; 2.4484x over previous
import functools

import jax
import jax.numpy as jnp
from jax import lax
from jax.experimental import pallas as pl
from jax.experimental.pallas import tpu as pltpu
from jax.experimental.pallas import tpu_sc as plsc

NC = 2
NS = 16
NW = NC * NS
L = 16


def _make_sc_kernel(B, S, D, P):
    BPW = B // NW
    assert B % NW == 0 and D % L == 0 and S % 2 == 0

    mesh = plsc.VectorSubcoreMesh(core_axis_name="c", subcore_axis_name="s")

    @functools.partial(
        pl.kernel,
        out_type=jax.ShapeDtypeStruct((B, S * D), jnp.float32),
        mesh=mesh,
        compiler_params=pltpu.CompilerParams(use_tc_tiling_on_sc=False,
                                             needs_layout_passes=False),
        scratch_types=[
            pltpu.VMEM((S, BPW), jnp.int32),
            pltpu.VMEM((P * D,), jnp.float32),
            pltpu.VMEM((BPW, D), jnp.float32),
            pltpu.VMEM((BPW, D), jnp.float32),
            pltpu.VMEM((BPW, D), jnp.float32),
            pltpu.VMEM((BPW, D), jnp.float32),
            pltpu.SemaphoreType.DMA,
            pltpu.SemaphoreType.DMA,
            pltpu.SemaphoreType.DMA,
            pltpu.SemaphoreType.DMA,
        ],
    )
    def sc_kernel(xT_hbm, posf_hbm, tok_hbm, out_hbm,
                  idx_v, pos_v, inA, inB, outA, outB,
                  gsemA, gsemB, osemA, osemB):
        wid = lax.axis_index("s") * NC + lax.axis_index("c")
        b0 = wid * BPW

        pltpu.sync_copy(xT_hbm.at[:, pl.ds(b0, BPW)], idx_v)
        pltpu.sync_copy(posf_hbm, pos_v)

        lanes = lax.iota(jnp.int32, L)

        def start_gather(s, inbuf, gsem):
            pltpu.async_copy(tok_hbm.at[idx_v.at[s]], inbuf, gsem)

        def gather_wait(inbuf, gsem):
            pltpu.make_async_copy(tok_hbm.at[idx_v.at[0]], inbuf, gsem).wait()

        def store_wait(outbuf, osem):
            pltpu.make_async_copy(
                outbuf, out_hbm.at[pl.ds(b0, BPW), pl.ds(0, D)], osem).wait()

        def compute(s, inbuf, outbuf):
            pbase = (s + 1) * D
            prow = [plsc.load_gather(pos_v, [pbase + k * L + lanes])
                    for k in range(D // L)]
            svec = jnp.full((L,), s, dtype=jnp.int32)
            for r in range(BPW):
                xspl = plsc.load_gather(
                    idx_v, [svec, jnp.full((L,), r, dtype=jnp.int32)])
                m = jnp.where(xspl > 0, jnp.float32(1.0), jnp.float32(0.0))
                for k in range(D // L):
                    sl = pl.ds(k * L, L)
                    outbuf[r, sl] = inbuf[r, sl] + prow[k] * m

        def start_store(s, outbuf, osem):
            pltpu.async_copy(
                outbuf, out_hbm.at[pl.ds(b0, BPW), pl.ds(s * D, D)], osem)

        start_gather(0, inA, gsemA)
        start_gather(1, inB, gsemB)

        def body(it, carry):
            s0 = 2 * it
            @pl.when(it > 0)
            def _():
                store_wait(outA, osemA)
            gather_wait(inA, gsemA)
            compute(s0, inA, outA)
            start_store(s0, outA, osemA)

            @pl.when(it < S // 2 - 1)
            def _():
                start_gather(s0 + 2, inA, gsemA)

            @pl.when(it > 0)
            def _():
                store_wait(outB, osemB)
            gather_wait(inB, gsemB)
            compute(s0 + 1, inB, outB)
            start_store(s0 + 1, outB, osemB)

            @pl.when(it < S // 2 - 1)
            def _():
                start_gather(s0 + 3, inB, gsemB)
            return carry

        lax.fori_loop(0, S // 2, body, 0)
        store_wait(outA, osemA)
        store_wait(outB, osemB)

    return sc_kernel


@jax.jit
def kernel(x, tok_table, pos_table):
    B, S = x.shape
    V, D = tok_table.shape
    P = pos_table.shape[0]
    xT = jnp.transpose(x)
    posf = pos_table.reshape(P * D)
    out = _make_sc_kernel(B, S, D, P)(xT, posf, tok_table)
    return out.reshape(B, S, D)
